# Initial kernel scaffold; baseline (speedup 1.0000x reference)
import jax
import jax.numpy as jnp
from jax.experimental import pallas as pl


def kernel(labels, adversaries):
    raise NotImplementedError("write your pallas kernel here")


# speedup vs baseline: 1.1089x; 1.1089x over previous
import functools

import jax
import jax.numpy as jnp
from jax import lax
from jax.experimental import pallas as pl
from jax.experimental.pallas import tpu as pltpu
from jax.experimental.pallas import tpu_sc as plsc

N_LABELS = 1000
N_DEPTHS = 3
BATCH = 16384
NC = 2
NS = 16
NW = NC * NS
BPW = BATCH // NW
CHUNK = 64
NCHUNK = BPW // CHUNK


def _sc_body(table_hbm, idx_hbm, o0, o1, o2, idx_v, buf, sem):
    wid = lax.axis_index("s") * NC + lax.axis_index("c")
    base = wid * BPW
    for d in range(N_DEPTHS):
        pltpu.sync_copy(
            idx_hbm.at[pl.ds(d * BATCH + base, BPW)],
            idx_v.at[pl.ds(d * BPW, BPW)],
        )
    outs = (o0, o1, o2)
    for d in range(N_DEPTHS):
        for c in range(NCHUNK):
            pltpu.async_copy(
                table_hbm.at[idx_v.at[pl.ds(d * BPW + c * CHUNK, CHUNK)]],
                buf,
                sem,
            ).wait()
            pltpu.sync_copy(buf, outs[d].at[pl.ds(base + c * CHUNK, CHUNK)])


@jax.jit
def kernel(labels, adversaries):
    leaf = labels[:, -1].astype(jnp.int32)
    idx_all = (
        leaf[None, :] + (N_LABELS * jnp.arange(N_DEPTHS, dtype=jnp.int32))[:, None]
    ).reshape(N_DEPTHS * BATCH)
    table = adversaries.reshape(N_DEPTHS * N_LABELS, N_LABELS)

    mesh = plsc.VectorSubcoreMesh(
        core_axis_name="c", subcore_axis_name="s", num_cores=NC, num_subcores=NS
    )
    out_sds = jax.ShapeDtypeStruct((BATCH, N_LABELS), jnp.float32)
    run = pl.kernel(
        _sc_body,
        out_type=(out_sds, out_sds, out_sds),
        mesh=mesh,
        scratch_types=[
            pltpu.VMEM((N_DEPTHS * BPW,), jnp.int32),
            pltpu.VMEM((CHUNK, N_LABELS), jnp.float32),
            pltpu.SemaphoreType.DMA,
        ],
        compiler_params=pltpu.CompilerParams(use_tc_tiling_on_sc=False),
    )
    return tuple(run(table, idx_all))


# speedup vs baseline: 1.1372x; 1.0255x over previous
import functools

import jax
import jax.numpy as jnp
from jax import lax
from jax.experimental import pallas as pl
from jax.experimental.pallas import tpu as pltpu
from jax.experimental.pallas import tpu_sc as plsc

N_LABELS = 1000
N_DEPTHS = 3
BATCH = 16384
NC = 2
NS = 16
NW = NC * NS
BPW = BATCH // NW
CHUNK = 64
NCHUNK = BPW // CHUNK


NBUF = 2
TOT = N_DEPTHS * NCHUNK


def _sc_body(table_hbm, idx_hbm, o0, o1, o2, idx_v, buf, gsem, wsem):
    wid = lax.axis_index("s") * NC + lax.axis_index("c")
    base = wid * BPW
    for d in range(N_DEPTHS):
        pltpu.sync_copy(
            idx_hbm.at[pl.ds(d * BATCH + base, BPW)],
            idx_v.at[pl.ds(d * BPW, BPW)],
        )
    outs = (o0, o1, o2)

    def gather_start(i, k):
        d, c = divmod(i, NCHUNK)
        return pltpu.async_copy(
            table_hbm.at[idx_v.at[pl.ds(d * BPW + c * CHUNK, CHUNK)]],
            buf.at[k],
            gsem.at[k],
        )

    def write_start(i, k):
        d, c = divmod(i, NCHUNK)
        return pltpu.async_copy(
            buf.at[k],
            outs[d].at[pl.ds(base + c * CHUNK, CHUNK)],
            wsem.at[k],
        )

    gh = [None] * NBUF
    wh = [None] * NBUF
    for i in range(TOT):
        k = i % NBUF
        if i >= NBUF:
            wh[k].wait()
        gh[k] = gather_start(i, k)
        if i >= 1:
            kp = (i - 1) % NBUF
            gh[kp].wait()
            wh[kp] = write_start(i - 1, kp)
    klast = (TOT - 1) % NBUF
    gh[klast].wait()
    wh[klast] = write_start(TOT - 1, klast)
    for k in range(NBUF):
        wh[k].wait()


@jax.jit
def kernel(labels, adversaries):
    leaf = labels[:, -1].astype(jnp.int32)
    idx_all = (
        leaf[None, :] + (N_LABELS * jnp.arange(N_DEPTHS, dtype=jnp.int32))[:, None]
    ).reshape(N_DEPTHS * BATCH)
    table = adversaries.reshape(N_DEPTHS * N_LABELS, N_LABELS)

    mesh = plsc.VectorSubcoreMesh(
        core_axis_name="c", subcore_axis_name="s", num_cores=NC, num_subcores=NS
    )
    out_sds = jax.ShapeDtypeStruct((BATCH, N_LABELS), jnp.float32)
    run = pl.kernel(
        _sc_body,
        out_type=(out_sds, out_sds, out_sds),
        mesh=mesh,
        scratch_types=[
            pltpu.VMEM((N_DEPTHS * BPW,), jnp.int32),
            pltpu.VMEM((NBUF, CHUNK, N_LABELS), jnp.float32),
            pltpu.SemaphoreType.DMA((NBUF,)),
            pltpu.SemaphoreType.DMA((NBUF,)),
        ],
        compiler_params=pltpu.CompilerParams(use_tc_tiling_on_sc=False),
    )
    return tuple(run(table, idx_all))


# speedup vs baseline: 1.1379x; 1.0006x over previous
import functools

import jax
import jax.numpy as jnp
from jax import lax
from jax.experimental import layout as jex_layout
from jax.experimental import pallas as pl
from jax.experimental.pallas import tpu as pltpu
from jax.experimental.pallas import tpu_sc as plsc

N_LABELS = 1000
N_DEPTHS = 3
BATCH = 16384
NC = 2
NS = 16
NW = NC * NS
BPW = BATCH // NW
CHUNK = 64
NCHUNK = BPW // CHUNK


NBUF = 2
TOT = N_DEPTHS * NCHUNK


def _sc_body(table_hbm, idx_hbm, o0, o1, o2, idx_v, buf, gsem, wsem):
    wid = lax.axis_index("s") * NC + lax.axis_index("c")
    base = wid * BPW
    for d in range(N_DEPTHS):
        pltpu.sync_copy(
            idx_hbm.at[pl.ds(d * BATCH + base, BPW)],
            idx_v.at[pl.ds(d * BPW, BPW)],
        )
    outs = (o0, o1, o2)

    def gather_start(i, k):
        d, c = divmod(i, NCHUNK)
        return pltpu.async_copy(
            table_hbm.at[idx_v.at[pl.ds(d * BPW + c * CHUNK, CHUNK)]],
            buf.at[k],
            gsem.at[k],
        )

    def write_start(i, k):
        d, c = divmod(i, NCHUNK)
        return pltpu.async_copy(
            buf.at[k],
            outs[d].at[pl.ds(base + c * CHUNK, CHUNK)],
            wsem.at[k],
        )

    gh = [None] * NBUF
    wh = [None] * NBUF
    for i in range(TOT):
        k = i % NBUF
        if i >= NBUF:
            wh[k].wait()
        gh[k] = gather_start(i, k)
        if i >= 1:
            kp = (i - 1) % NBUF
            gh[kp].wait()
            wh[kp] = write_start(i - 1, kp)
    klast = (TOT - 1) % NBUF
    gh[klast].wait()
    wh[klast] = write_start(TOT - 1, klast)
    for k in range(NBUF):
        wh[k].wait()


@functools.lru_cache(maxsize=None)
def _jitted(dev):
    fmt = jex_layout.Format(
        jex_layout.Layout(major_to_minor=(0, 1), tiling=()),
        jax.sharding.SingleDeviceSharding(dev),
    )
    return jax.jit(_impl, out_shardings=(fmt, fmt, fmt))


def kernel(labels, adversaries):
    return _jitted(jax.devices()[0])(labels, adversaries)


def _impl(labels, adversaries):
    leaf = labels[:, -1].astype(jnp.int32)
    idx_all = (
        leaf[None, :] + (N_LABELS * jnp.arange(N_DEPTHS, dtype=jnp.int32))[:, None]
    ).reshape(N_DEPTHS * BATCH)
    table = adversaries.reshape(N_DEPTHS * N_LABELS, N_LABELS)

    mesh = plsc.VectorSubcoreMesh(
        core_axis_name="c", subcore_axis_name="s", num_cores=NC, num_subcores=NS
    )
    out_sds = jax.ShapeDtypeStruct((BATCH, N_LABELS), jnp.float32)
    run = pl.kernel(
        _sc_body,
        out_type=(out_sds, out_sds, out_sds),
        mesh=mesh,
        scratch_types=[
            pltpu.VMEM((N_DEPTHS * BPW,), jnp.int32),
            pltpu.VMEM((NBUF, CHUNK, N_LABELS), jnp.float32),
            pltpu.SemaphoreType.DMA((NBUF,)),
            pltpu.SemaphoreType.DMA((NBUF,)),
        ],
        compiler_params=pltpu.CompilerParams(use_tc_tiling_on_sc=False),
    )
    return tuple(run(table, idx_all))
